# Initial kernel scaffold; baseline (speedup 1.0000x reference)
import jax
import jax.numpy as jnp
from jax.experimental import pallas as pl


def kernel(x, edge_index, edge_attr, batch, node_emb, edge_emb, conv_w1, conv_b1, conv_w2, conv_b2, mlp_w1, mlp_b1, mlp_w2, mlp_b2, mlp_w3, mlp_b3):
    raise NotImplementedError("write your pallas kernel here")


# speedup vs baseline: 3.8893x; 3.8893x over previous
import functools

import jax
import jax.numpy as jnp
from jax import lax
from jax.experimental import pallas as pl
from jax.experimental.pallas import tpu as pltpu
from jax.experimental.pallas import tpu_sc as plsc

_NC = 2
_NS = 16
_K = 128


def _make_sc_scatter(N, E, Ch):
    ECH = E // _K
    EPT = -(-ECH // _NS)
    RCH = 200
    RNCH = N // RCH
    RPT = -(-RNCH // _NS)

    mesh = plsc.VectorSubcoreMesh(core_axis_name="c", subcore_axis_name="s")

    @functools.partial(
        pl.kernel,
        out_type=jax.ShapeDtypeStruct((2 * N, Ch), jnp.float32),
        mesh=mesh,
        scratch_types=[
            pltpu.VMEM((_K,), jnp.int32),
            pltpu.VMEM((_K,), jnp.int32),
            pltpu.VMEM((_K, Ch), jnp.float32),
            pltpu.VMEM((RCH, Ch), jnp.float32),
            pltpu.VMEM_SHARED((N, Ch), jnp.float32),
            pltpu.SemaphoreType.DMA,
        ],
        compiler_params=pltpu.CompilerParams(use_tc_tiling_on_sc=False),
    )
    def sc_scatter(h_hbm, gsrc_hbm, dst_hbm, zero_hbm, out_hbm, sidx, didx,
                   rows, stage, shared, sem):
        c = lax.axis_index("c")
        s = lax.axis_index("s")

        pltpu.sync_copy(zero_hbm, stage)

        def init_chunk(j, carry):
            g = s + j * _NS

            @pl.when(g < RNCH)
            def _():
                pltpu.sync_copy(stage, shared.at[pl.ds(g * RCH, RCH)])

            return carry

        lax.fori_loop(0, RPT, init_chunk, 0)
        plsc.subcore_barrier()

        def chunk(j, carry):
            g = s + j * _NS

            @pl.when(g < ECH)
            def _():
                base = g * _K
                pltpu.sync_copy(gsrc_hbm.at[pl.ds(c * E + base, _K)], sidx)
                pltpu.sync_copy(dst_hbm.at[pl.ds(base, _K)], didx)
                pltpu.async_copy(h_hbm.at[sidx], rows, sem).wait()
                pltpu.sync_copy(rows, shared.at[didx], add=True)

            return carry

        lax.fori_loop(0, EPT, chunk, 0)
        plsc.subcore_barrier()

        def out_chunk(j, carry):
            g = s + j * _NS

            @pl.when(g < RNCH)
            def _():
                pltpu.sync_copy(shared.at[pl.ds(g * RCH, RCH)], stage)
                pltpu.sync_copy(stage, out_hbm.at[pl.ds(c * N + g * RCH, RCH)])

            return carry

        lax.fori_loop(0, RPT, out_chunk, 0)

    return sc_scatter


def _make_sc_pool(N, G, Ch):
    CHN = 80
    NCHUNK = N // CHN
    PER_TILE = -(-NCHUNK // _NS)
    GR = G // _NS

    mesh = plsc.VectorSubcoreMesh(core_axis_name="c", subcore_axis_name="s")

    @functools.partial(
        pl.kernel,
        out_type=jax.ShapeDtypeStruct((2 * G, Ch), jnp.float32),
        mesh=mesh,
        scratch_types=[
            pltpu.VMEM((CHN,), jnp.int32),
            pltpu.VMEM((CHN, Ch), jnp.float32),
            pltpu.VMEM((G // 4, Ch), jnp.float32),
            pltpu.VMEM_SHARED((G, Ch), jnp.float32),
        ],
        compiler_params=pltpu.CompilerParams(use_tc_tiling_on_sc=False),
    )
    def sc_pool(h_hbm, batch_hbm, zero_hbm, out_hbm, bidx, rows, stage, shared):
        c = lax.axis_index("c")
        s = lax.axis_index("s")

        @pl.when(s < 4)
        def _():
            q = G // 4
            pltpu.sync_copy(zero_hbm.at[pl.ds(s * q, q)], stage)
            pltpu.sync_copy(stage, shared.at[pl.ds(s * q, q)])

        plsc.subcore_barrier()

        def chunk(j, carry):
            g = s + j * _NS

            @pl.when(g < NCHUNK)
            def _():
                pltpu.sync_copy(h_hbm.at[pl.ds(c * N + g * CHN, CHN)], rows)
                pltpu.sync_copy(batch_hbm.at[pl.ds(g * CHN, CHN)], bidx)
                pltpu.sync_copy(rows, shared.at[bidx], add=True)

            return carry

        lax.fori_loop(0, PER_TILE, chunk, 0)
        plsc.subcore_barrier()
        pltpu.sync_copy(shared.at[pl.ds(s * GR, GR)], stage.at[pl.ds(0, GR)])
        pltpu.sync_copy(stage.at[pl.ds(0, GR)],
                        out_hbm.at[pl.ds(c * G + s * GR, GR)])

    return sc_pool


def _embed_kernel(x_ref, emb_ref, out_ref):
    xb = x_ref[...]
    r = xb.shape[0]
    v = emb_ref.shape[0]
    onehot = (xb[:, None] == lax.broadcasted_iota(jnp.int32, (r, v), 1))
    onehot = onehot.astype(jnp.float32)
    h = jnp.dot(onehot, emb_ref[...], preferred_element_type=jnp.float32,
                precision=lax.Precision.HIGHEST)
    ch = out_ref.shape[2]
    out_ref[0] = h[:, :ch]
    out_ref[1] = h[:, ch:]


def _dense_kernel(h_ref, a_ref, w1_ref, b1_ref, w2_ref, b2_ref, out_ref):
    ch = h_ref.shape[2]
    h_in = jnp.concatenate([h_ref[0], h_ref[1]], axis=1)
    agg = jnp.concatenate([a_ref[0], a_ref[1]], axis=1)
    z = h_in + agg
    z1 = jnp.dot(z, w1_ref[...], preferred_element_type=jnp.float32) + b1_ref[...]
    z1 = jnp.maximum(z1, 0.0)
    h = jnp.dot(z1, w2_ref[...], preferred_element_type=jnp.float32) + b2_ref[...]
    out_ref[0] = h[:, :ch]
    out_ref[1] = h[:, ch:]


def _mlp_kernel(p_ref, w1_ref, b1_ref, w2_ref, b2_ref, w3_ref, b3_ref, out_ref):
    p = jnp.concatenate([p_ref[0], p_ref[1]], axis=1)
    z1 = jnp.dot(p, w1_ref[...], preferred_element_type=jnp.float32) + b1_ref[...]
    z1 = jnp.maximum(z1, 0.0)
    z2 = jnp.dot(z1, w2_ref[...], preferred_element_type=jnp.float32) + b2_ref[...]
    z2 = jnp.maximum(z2, 0.0)
    out_ref[...] = (jnp.dot(z2, w3_ref[...], preferred_element_type=jnp.float32)
                    + b3_ref[...])


def kernel(x, edge_index, edge_attr, batch, node_emb, edge_emb,
           conv_w1, conv_b1, conv_w2, conv_b2,
           mlp_w1, mlp_b1, mlp_w2, mlp_b2, mlp_w3, mlp_b3):
    N = x.shape[0]
    E = edge_index.shape[1]
    C = node_emb.shape[1]
    L = conv_w1.shape[0]
    G = 512
    Ch = C // 2
    R = 2048

    src = edge_index[0]
    dst = edge_index[1]
    gsrc = jnp.concatenate([src, src + N])
    zeros_g = jnp.zeros((G, Ch), jnp.float32)
    zeros_r = jnp.zeros((200, Ch), jnp.float32)

    grid = pl.cdiv(N, R)
    h = pl.pallas_call(
        _embed_kernel,
        grid=(grid,),
        in_specs=[
            pl.BlockSpec((R,), lambda i: (i,)),
            pl.BlockSpec((node_emb.shape[0], C), lambda i: (0, 0)),
        ],
        out_specs=pl.BlockSpec((2, R, Ch), lambda i: (0, i, 0)),
        out_shape=jax.ShapeDtypeStruct((2, N, Ch), jnp.float32),
    )(jnp.squeeze(x, -1), node_emb).reshape(2 * N, Ch)

    sc_scatter = _make_sc_scatter(N, E, Ch)
    dense = functools.partial(
        pl.pallas_call,
        _dense_kernel,
        grid=(grid,),
        in_specs=[
            pl.BlockSpec((2, R, Ch), lambda i: (0, i, 0)),
            pl.BlockSpec((2, R, Ch), lambda i: (0, i, 0)),
            pl.BlockSpec((C, C), lambda i: (0, 0)),
            pl.BlockSpec((1, C), lambda i: (0, 0)),
            pl.BlockSpec((C, C), lambda i: (0, 0)),
            pl.BlockSpec((1, C), lambda i: (0, 0)),
        ],
        out_specs=pl.BlockSpec((2, R, Ch), lambda i: (0, i, 0)),
        out_shape=jax.ShapeDtypeStruct((2, N, Ch), jnp.float32),
    )()

    for l in range(L):
        agg = sc_scatter(h, gsrc, dst, zeros_r)
        h = dense(h.reshape(2, N, Ch), agg.reshape(2, N, Ch),
                  conv_w1[l], conv_b1[l][None, :],
                  conv_w2[l], conv_b2[l][None, :]).reshape(2 * N, Ch)

    sc_pool = _make_sc_pool(N, G, Ch)
    pooled = sc_pool(h, batch, zeros_g)

    out = pl.pallas_call(
        _mlp_kernel,
        in_specs=[
            pl.BlockSpec((2, G, Ch), lambda: (0, 0, 0)),
            pl.BlockSpec(mlp_w1.shape, lambda: (0, 0)),
            pl.BlockSpec((1, mlp_b1.shape[0]), lambda: (0, 0)),
            pl.BlockSpec(mlp_w2.shape, lambda: (0, 0)),
            pl.BlockSpec((1, mlp_b2.shape[0]), lambda: (0, 0)),
            pl.BlockSpec(mlp_w3.shape, lambda: (0, 0)),
            pl.BlockSpec((1, 1), lambda: (0, 0)),
        ],
        out_specs=pl.BlockSpec((G, 1), lambda: (0, 0)),
        out_shape=jax.ShapeDtypeStruct((G, 1), jnp.float32),
    )(pooled.reshape(2, G, Ch), mlp_w1, mlp_b1[None, :], mlp_w2,
      mlp_b2[None, :], mlp_w3, mlp_b3[None, :])
    return out
